# baseline (speedup 1.0000x reference)
import functools

import jax
import jax.numpy as jnp
from jax.experimental import pallas as pl
from jax.experimental.pallas import tpu as pltpu

TOP_K = 2
NUM_EXPERTS = 16
D_MODEL = 2048
TBLK = 2048
DBLK = 512
NDB = D_MODEL // DBLK


def _router_body(x_ref, w_ref, b_ref, out_ref, acc_ref):
    j = pl.program_id(1)
    part = jnp.dot(x_ref[...], w_ref[...], preferred_element_type=jnp.float32)

    @pl.when(j == 0)
    def _init():
        acc_ref[...] = part + b_ref[...]

    @pl.when(j > 0)
    def _acc():
        acc_ref[...] += part

    @pl.when(j == NDB - 1)
    def _finish():
        logits = acc_ref[...]
        lmax = jnp.max(logits, axis=-1, keepdims=True)
        e = jnp.exp(logits - lmax)
        scores = e / jnp.sum(e, axis=-1, keepdims=True)
        idx = jax.lax.broadcasted_iota(jnp.int32, scores.shape, 1)
        m1 = jnp.max(scores, axis=-1, keepdims=True)
        i1 = jnp.min(jnp.where(scores == m1, idx, NUM_EXPERTS), axis=-1,
                     keepdims=True)
        sel1 = idx == i1
        s2 = jnp.where(sel1, -jnp.inf, scores)
        m2 = jnp.max(s2, axis=-1, keepdims=True)
        i2 = jnp.min(jnp.where(s2 == m2, idx, NUM_EXPERTS), axis=-1,
                     keepdims=True)
        sel2 = idx == i2
        out_ref[...] = jnp.where(sel1 | sel2, scores, 0.0)


@jax.jit
def kernel(x, W, b):
    B, S, D = x.shape
    E = W.shape[1]
    T = B * S
    xf = x.reshape(T, D)
    bf = b.reshape(1, E)
    out = pl.pallas_call(
        _router_body,
        grid=(T // TBLK, NDB),
        in_specs=[
            pl.BlockSpec((TBLK, DBLK), lambda i, j: (i, j)),
            pl.BlockSpec((DBLK, E), lambda i, j: (j, 0)),
            pl.BlockSpec((1, E), lambda i, j: (0, 0)),
        ],
        out_specs=pl.BlockSpec((TBLK, E), lambda i, j: (i, 0)),
        out_shape=jax.ShapeDtypeStruct((T, E), jnp.float32),
        scratch_shapes=[pltpu.VMEM((TBLK, E), jnp.float32)],
        compiler_params=pltpu.CompilerParams(
            dimension_semantics=("parallel", "arbitrary"),
        ),
    )(xf, W, bf)
    return out.reshape(B, S, E)


# speedup vs baseline: 1.7542x; 1.7542x over previous
import functools

import jax
import jax.numpy as jnp
from jax.experimental import pallas as pl
from jax.experimental.pallas import tpu as pltpu

TOP_K = 2
NUM_EXPERTS = 16
D_MODEL = 2048
TBLK = 2048


def _router_body(x_ref, wt_ref, b_ref, out_ref):
    logits = jax.lax.dot_general(
        wt_ref[...], x_ref[...],
        dimension_numbers=(((1,), (1,)), ((), ())),
        preferred_element_type=jnp.float32,
    )
    logits = logits + b_ref[...]
    lmax = jnp.max(logits, axis=0, keepdims=True)
    e = jnp.exp(logits - lmax)
    scores = e / jnp.sum(e, axis=0, keepdims=True)
    idx = jax.lax.broadcasted_iota(jnp.int32, scores.shape, 0)
    m1 = jnp.max(scores, axis=0, keepdims=True)
    i1 = jnp.min(jnp.where(scores == m1, idx, NUM_EXPERTS), axis=0,
                 keepdims=True)
    sel1 = idx == i1
    s2 = jnp.where(sel1, -jnp.inf, scores)
    m2 = jnp.max(s2, axis=0, keepdims=True)
    i2 = jnp.min(jnp.where(s2 == m2, idx, NUM_EXPERTS), axis=0,
                 keepdims=True)
    sel2 = idx == i2
    out_ref[...] = jnp.where(sel1 | sel2, scores, 0.0)


@jax.jit
def kernel(x, W, b):
    B, S, D = x.shape
    E = W.shape[1]
    T = B * S
    xf = x.reshape(T, D)
    wt = W.T
    bf = b.reshape(E, 1)
    out = pl.pallas_call(
        _router_body,
        grid=(T // TBLK,),
        in_specs=[
            pl.BlockSpec((TBLK, D), lambda i: (i, 0)),
            pl.BlockSpec((E, D), lambda i: (0, 0)),
            pl.BlockSpec((E, 1), lambda i: (0, 0)),
        ],
        out_specs=pl.BlockSpec((E, TBLK), lambda i: (0, i)),
        out_shape=jax.ShapeDtypeStruct((E, T), jnp.float32),
        compiler_params=pltpu.CompilerParams(
            dimension_semantics=("arbitrary",),
        ),
    )(xf, wt, bf)
    return out.T.reshape(B, S, E)


# speedup vs baseline: 1.8171x; 1.0359x over previous
import functools

import jax
import jax.numpy as jnp
from jax.experimental import pallas as pl
from jax.experimental.pallas import tpu as pltpu

TOP_K = 2
NUM_EXPERTS = 16
D_MODEL = 2048
TBLK = 1024


def _router_body(x_ref, wt_ref, b_ref, out_ref):
    logits = jax.lax.dot_general(
        wt_ref[...], x_ref[...],
        dimension_numbers=(((1,), (1,)), ((), ())),
        preferred_element_type=jnp.float32,
    )
    logits = logits + b_ref[...]
    lmax = jnp.max(logits, axis=0, keepdims=True)
    e = jnp.exp(logits - lmax)
    scores = e / jnp.sum(e, axis=0, keepdims=True)
    idx = jax.lax.broadcasted_iota(jnp.int32, scores.shape, 0)
    m1 = jnp.max(scores, axis=0, keepdims=True)
    i1 = jnp.min(jnp.where(scores == m1, idx, NUM_EXPERTS), axis=0,
                 keepdims=True)
    sel1 = idx == i1
    s2 = jnp.where(sel1, -jnp.inf, scores)
    m2 = jnp.max(s2, axis=0, keepdims=True)
    i2 = jnp.min(jnp.where(s2 == m2, idx, NUM_EXPERTS), axis=0,
                 keepdims=True)
    sel2 = idx == i2
    out_ref[...] = jnp.where(sel1 | sel2, scores, 0.0)


@jax.jit
def kernel(x, W, b):
    B, S, D = x.shape
    E = W.shape[1]
    T = B * S
    xf = x.reshape(T, D)
    wt = W.T
    bf = b.reshape(E, 1)
    out = pl.pallas_call(
        _router_body,
        grid=(T // TBLK,),
        in_specs=[
            pl.BlockSpec((TBLK, D), lambda i: (i, 0)),
            pl.BlockSpec((E, D), lambda i: (0, 0)),
            pl.BlockSpec((E, 1), lambda i: (0, 0)),
        ],
        out_specs=pl.BlockSpec((E, TBLK), lambda i: (0, i)),
        out_shape=jax.ShapeDtypeStruct((E, T), jnp.float32),
        compiler_params=pltpu.CompilerParams(
            dimension_semantics=("arbitrary",),
        ),
    )(xf, wt, bf)
    return out.T.reshape(B, S, E)
